# Initial kernel scaffold; baseline (speedup 1.0000x reference)
import jax
import jax.numpy as jnp
from jax.experimental import pallas as pl


def kernel(X, edge_index, edge_weight, previous_indices, sampled_nodes, fs_w, W1, b1, W2, b2, W_out, b_out):
    raise NotImplementedError("write your pallas kernel here")


# speedup vs baseline: 2.1968x; 2.1968x over previous
import functools

import jax
import jax.numpy as jnp
from jax import lax
from jax.experimental import pallas as pl
from jax.experimental.pallas import tpu as pltpu
from jax.experimental.pallas import tpu_sc as plsc

_N = 10000
_NPAD = 10240
_E = 160000
_EPAD = 163840
_K = 128
_D = 128
_NS = 16
_NC = 2
_ROWS_PER_SUB = _NPAD // _NS
_ROW_CHUNKS = _ROWS_PER_SUB // _K
_EDGES_PER_SUB = _EPAD // _NS
_EDGE_CHUNKS = _EDGES_PER_SUB // _K


def _sc_aggregate(xh, prevp, srcp, dstp, wp):
    mesh = plsc.VectorSubcoreMesh(core_axis_name="c", subcore_axis_name="s")

    @functools.partial(
        pl.kernel,
        mesh=mesh,
        out_type=jax.ShapeDtypeStruct((_NC * _NPAD, _D), jnp.float32),
        scratch_types=[
            pltpu.VMEM_SHARED((_NPAD, _D), jnp.float32),
            pltpu.VMEM((_K,), jnp.int32),
            pltpu.VMEM((_K,), jnp.int32),
            pltpu.VMEM((_K,), jnp.float32),
            pltpu.VMEM((_K, _D), jnp.float32),
            pltpu.SemaphoreType.DMA,
        ],
    )
    def k(xh_hbm, prev_hbm, src_hbm, dst_hbm, w_hbm, out_hbm,
          acc, ibuf, dbuf, wbuf, rows, sem):
        c = lax.axis_index("c")
        s = lax.axis_index("s")

        def p1(j, carry):
            base = s * _ROWS_PER_SUB + j * _K
            pltpu.sync_copy(prev_hbm.at[pl.ds(base, _K)], ibuf)
            for t in range(_K // 16):
                v = ibuf[pl.ds(t * 16, 16)]
                ibuf[pl.ds(t * 16, 16)] = v * 2 + c
            pltpu.async_copy(xh_hbm.at[ibuf], rows, sem).wait()
            pltpu.sync_copy(rows, acc.at[pl.ds(base, _K)])
            return carry

        lax.fori_loop(0, _ROW_CHUNKS, p1, 0)
        plsc.subcore_barrier()

        def p2(j, carry):
            base = s * _EDGES_PER_SUB + j * _K
            pltpu.sync_copy(src_hbm.at[pl.ds(base, _K)], ibuf)
            pltpu.sync_copy(dst_hbm.at[pl.ds(base, _K)], dbuf)
            pltpu.sync_copy(w_hbm.at[pl.ds(base, _K)], wbuf)
            for t in range(_K // 16):
                v = ibuf[pl.ds(t * 16, 16)]
                ibuf[pl.ds(t * 16, 16)] = v * 2 + c
            pltpu.async_copy(xh_hbm.at[ibuf], rows, sem).wait()

            def pm(g, inner):
                wv = wbuf[pl.ds(g * 16, 16)]
                for l in range(16):
                    w = wv[l]
                    e = g * 16 + l
                    for t in range(_D // 16):
                        rows[e, pl.ds(t * 16, 16)] = rows[e, pl.ds(t * 16, 16)] * w
                return inner

            lax.fori_loop(0, _K // 16, pm, 0)
            pltpu.sync_copy(rows, acc.at[dbuf], add=True)
            return carry

        lax.fori_loop(0, _EDGE_CHUNKS, p2, 0)
        plsc.subcore_barrier()

        def p3(j, carry):
            base = s * _ROWS_PER_SUB + j * _K
            pltpu.sync_copy(acc.at[pl.ds(base, _K)],
                            out_hbm.at[pl.ds(c * _NPAD + base, _K)])
            return carry

        lax.fori_loop(0, _ROW_CHUNKS, p3, 0)

    return k(xh, prevp, srcp, dstp, wp)


def _tc_head(hsc, fs_w, W2, b2, W_out, b_out):
    BN = 512
    nblk = _NPAD // BN
    fsw2 = fs_w.reshape(256, 1)
    b2r = b2.reshape(1, 128)
    woutp = jnp.zeros((128, 128), jnp.float32).at[:, :40].set(W_out)
    boutp = jnp.zeros((1, 128), jnp.float32).at[0, :40].set(b_out)

    def body(h0_ref, h1_ref, g_ref, w2_ref, b2_ref, wo_ref, bo_ref, o_ref):
        g = 0.5 * jax.nn.sigmoid(g_ref[...])
        w2p = w2_ref[...] * g
        x = jnp.dot(h0_ref[...], w2p[:128, :], preferred_element_type=jnp.float32)
        x = x + jnp.dot(h1_ref[...], w2p[128:, :], preferred_element_type=jnp.float32)
        x = jnp.maximum(x + b2_ref[...], 0.0)
        o_ref[...] = jnp.dot(x, wo_ref[...], preferred_element_type=jnp.float32) + bo_ref[...]

    out = pl.pallas_call(
        body,
        grid=(nblk,),
        in_specs=[
            pl.BlockSpec((BN, _D), lambda i: (i, 0)),
            pl.BlockSpec((BN, _D), lambda i: (i + nblk, 0)),
            pl.BlockSpec((256, 1), lambda i: (0, 0)),
            pl.BlockSpec((256, 128), lambda i: (0, 0)),
            pl.BlockSpec((1, 128), lambda i: (0, 0)),
            pl.BlockSpec((128, 128), lambda i: (0, 0)),
            pl.BlockSpec((1, 128), lambda i: (0, 0)),
        ],
        out_specs=pl.BlockSpec((BN, 128), lambda i: (i, 0)),
        out_shape=jax.ShapeDtypeStruct((_NPAD, 128), jnp.float32),
    )(hsc, hsc, fsw2, W2, b2r, woutp, boutp)
    return out[:_N, :40]


def kernel(X, edge_index, edge_weight, previous_indices, sampled_nodes,
           fs_w, W1, b1, W2, b2, W_out, b_out):
    xh = X.reshape(2 * _N, _D)
    zi = jnp.zeros((_NPAD - _N,), jnp.int32)
    ze = jnp.zeros((_EPAD - _E,), jnp.int32)
    prevp = jnp.concatenate([previous_indices[2], zi])
    srcp = jnp.concatenate([edge_index[2, 0], ze])
    dstp = jnp.concatenate([edge_index[2, 1], ze])
    wp = jnp.concatenate([edge_weight[2], ze.astype(jnp.float32)])
    hsc = _sc_aggregate(xh, prevp, srcp, dstp, wp)
    return _tc_head(hsc, fs_w, W2, b2, W_out, b_out)


# speedup vs baseline: 3.1985x; 1.4559x over previous
import functools

import jax
import jax.numpy as jnp
from jax import lax
from jax.experimental import pallas as pl
from jax.experimental.pallas import tpu as pltpu
from jax.experimental.pallas import tpu_sc as plsc

_N = 10000
_NPAD = 10240
_E = 160000
_EPAD = 163840
_K = 128
_D = 128
_NS = 16
_NC = 2
_ROWS_PER_SUB = _NPAD // _NS
_ROW_CHUNKS = _ROWS_PER_SUB // _K
_EDGES_PER_SUB = _EPAD // _NS
_EDGE_CHUNKS = _EDGES_PER_SUB // _K


def _sc_aggregate(xh, prevp, srcp, dstp, wp):
    mesh = plsc.VectorSubcoreMesh(core_axis_name="c", subcore_axis_name="s")
    nseg = 5
    seg = _EDGE_CHUNKS // nseg

    @functools.partial(
        pl.kernel,
        mesh=mesh,
        out_type=jax.ShapeDtypeStruct((_NC * _NPAD, _D), jnp.float32),
        scratch_types=[
            pltpu.VMEM_SHARED((_NPAD, _D), jnp.float32),
            pltpu.VMEM((_ROW_CHUNKS, _K), jnp.int32),
            pltpu.VMEM((seg, _K), jnp.int32),
            pltpu.VMEM((seg, _K), jnp.int32),
            pltpu.VMEM((seg, _K), jnp.float32),
            pltpu.VMEM((_K, _D), jnp.float32),
            pltpu.VMEM((_K, _D), jnp.float32),
            pltpu.SemaphoreType.DMA,
            pltpu.SemaphoreType.DMA,
        ],
    )
    def k(xh_hbm, prev_hbm, src_hbm, dst_hbm, w_hbm, out_hbm,
          acc, pidx, sidx, didx, wseg, rows_a, rows_b, sem_a, sem_b):
        c = lax.axis_index("c")
        s = lax.axis_index("s")

        pltpu.sync_copy(prev_hbm.at[s], pidx)

        def tfp(r, carry):
            for t in range(_K // 16):
                v = pidx[r, pl.ds(t * 16, 16)]
                pidx[r, pl.ds(t * 16, 16)] = v * 2 + c
            return carry

        lax.fori_loop(0, _ROW_CHUNKS, tfp, 0)

        def p1(j, carry):
            base = s * _ROWS_PER_SUB + j * _K
            pltpu.async_copy(xh_hbm.at[pidx.at[j]], rows_a, sem_a).wait()
            pltpu.sync_copy(rows_a, acc.at[pl.ds(base, _K)])
            return carry

        lax.fori_loop(0, _ROW_CHUNKS, p1, 0)
        plsc.subcore_barrier()

        def mul(rows, j):
            def g_body(g, carry):
                wv = wseg[j, pl.ds(g * 16, 16)]
                for l in range(16):
                    w = wv[l]
                    e = g * 16 + l
                    for t in range(_D // 16):
                        rows[e, pl.ds(t * 16, 16)] = rows[e, pl.ds(t * 16, 16)] * w
                return carry

            lax.fori_loop(0, _K // 16, g_body, 0)

        def p2seg(gseg, carry):
            pltpu.sync_copy(src_hbm.at[s, gseg], sidx)
            pltpu.sync_copy(dst_hbm.at[s, gseg], didx)
            pltpu.sync_copy(w_hbm.at[s, gseg], wseg)

            def tfs(r, inner):
                for t in range(_K // 16):
                    v = sidx[r, pl.ds(t * 16, 16)]
                    sidx[r, pl.ds(t * 16, 16)] = v * 2 + c
                return inner

            lax.fori_loop(0, seg, tfs, 0)
            pltpu.async_copy(xh_hbm.at[sidx.at[0]], rows_a, sem_a)

            def pair(h, inner):
                j0 = 2 * h
                j1 = 2 * h + 1
                pltpu.async_copy(xh_hbm.at[sidx.at[j1]], rows_b, sem_b)
                pltpu.make_async_copy(xh_hbm.at[sidx.at[j0]], rows_a, sem_a).wait()
                mul(rows_a, j0)
                pltpu.sync_copy(rows_a, acc.at[didx.at[j0]], add=True)

                @pl.when(h + 1 < seg // 2)
                def _():
                    pltpu.async_copy(xh_hbm.at[sidx.at[j0 + 2]], rows_a, sem_a)

                pltpu.make_async_copy(xh_hbm.at[sidx.at[j1]], rows_b, sem_b).wait()
                mul(rows_b, j1)
                pltpu.sync_copy(rows_b, acc.at[didx.at[j1]], add=True)
                return inner

            lax.fori_loop(0, seg // 2, pair, 0)
            return carry

        lax.fori_loop(0, nseg, p2seg, 0)
        plsc.subcore_barrier()

        def p3(j, carry):
            base = s * _ROWS_PER_SUB + j * _K
            pltpu.sync_copy(acc.at[pl.ds(base, _K)],
                            out_hbm.at[pl.ds(c * _NPAD + base, _K)])
            return carry

        lax.fori_loop(0, _ROW_CHUNKS, p3, 0)

    return k(xh, prevp, srcp, dstp, wp)


def _tc_head(hsc, fs_w, W2, b2, W_out, b_out):
    BN = 512
    nblk = _NPAD // BN
    fsw2 = fs_w.reshape(256, 1)
    b2r = b2.reshape(1, 128)
    woutp = jnp.zeros((128, 128), jnp.float32).at[:, :40].set(W_out)
    boutp = jnp.zeros((1, 128), jnp.float32).at[0, :40].set(b_out)

    def body(h0_ref, h1_ref, g_ref, w2_ref, b2_ref, wo_ref, bo_ref, o_ref):
        g = 0.5 * jax.nn.sigmoid(g_ref[...])
        w2p = w2_ref[...] * g
        x = jnp.dot(h0_ref[...], w2p[:128, :], preferred_element_type=jnp.float32)
        x = x + jnp.dot(h1_ref[...], w2p[128:, :], preferred_element_type=jnp.float32)
        x = jnp.maximum(x + b2_ref[...], 0.0)
        o_ref[...] = jnp.dot(x, wo_ref[...], preferred_element_type=jnp.float32) + bo_ref[...]

    out = pl.pallas_call(
        body,
        grid=(nblk,),
        in_specs=[
            pl.BlockSpec((BN, _D), lambda i: (i, 0)),
            pl.BlockSpec((BN, _D), lambda i: (i + nblk, 0)),
            pl.BlockSpec((256, 1), lambda i: (0, 0)),
            pl.BlockSpec((256, 128), lambda i: (0, 0)),
            pl.BlockSpec((1, 128), lambda i: (0, 0)),
            pl.BlockSpec((128, 128), lambda i: (0, 0)),
            pl.BlockSpec((1, 128), lambda i: (0, 0)),
        ],
        out_specs=pl.BlockSpec((BN, 128), lambda i: (i, 0)),
        out_shape=jax.ShapeDtypeStruct((_NPAD, 128), jnp.float32),
    )(hsc, hsc, fsw2, W2, b2r, woutp, boutp)
    return out[:_N, :40]


def kernel(X, edge_index, edge_weight, previous_indices, sampled_nodes,
           fs_w, W1, b1, W2, b2, W_out, b_out):
    xh = X.reshape(2 * _N, _D)
    zi = jnp.zeros((_NPAD - _N,), jnp.int32)
    ze = jnp.zeros((_EPAD - _E,), jnp.int32)
    prevp = jnp.concatenate([previous_indices[2], zi]).reshape(_NS, _ROW_CHUNKS, _K)
    srcp = jnp.concatenate([edge_index[2, 0], ze]).reshape(_NS, 5, 16, _K)
    dstp = jnp.concatenate([edge_index[2, 1], ze]).reshape(_NS, 5, 16, _K)
    wp = jnp.concatenate([edge_weight[2], ze.astype(jnp.float32)]).reshape(
        _NS, 5, 16, _K)
    hsc = _sc_aggregate(xh, prevp, srcp, dstp, wp)
    return _tc_head(hsc, fs_w, W2, b2, W_out, b_out)


# speedup vs baseline: 3.3083x; 1.0343x over previous
import functools

import jax
import jax.numpy as jnp
from jax import lax
from jax.experimental import pallas as pl
from jax.experimental.pallas import tpu as pltpu
from jax.experimental.pallas import tpu_sc as plsc

_N = 10000
_NPAD = 10240
_E = 160000
_EPAD = 163840
_K = 64
_D = 128
_NS = 16
_NC = 2
_NBUF = 4
_ROWS_PER_SUB = _NPAD // _NS
_ROW_CHUNKS = _ROWS_PER_SUB // _K
_EDGES_PER_SUB = _EPAD // _NS
_NSEG = 5
_SEG = _EDGES_PER_SUB // _K // _NSEG


def _sc_aggregate(xh, prevp, srcp, dstp, wp):
    mesh = plsc.VectorSubcoreMesh(core_axis_name="c", subcore_axis_name="s")

    @functools.partial(
        pl.kernel,
        mesh=mesh,
        out_type=jax.ShapeDtypeStruct((_NC * _NPAD, _D), jnp.float32),
        scratch_types=[
            pltpu.VMEM_SHARED((_NPAD, _D), jnp.float32),
            pltpu.VMEM((_ROW_CHUNKS, _K), jnp.int32),
            pltpu.VMEM((_SEG, _K), jnp.int32),
            pltpu.VMEM((_SEG, _K), jnp.int32),
            pltpu.VMEM((_SEG, _K), jnp.float32),
            pltpu.VMEM((_K, _D), jnp.float32),
            pltpu.VMEM((_K, _D), jnp.float32),
            pltpu.VMEM((_K, _D), jnp.float32),
            pltpu.VMEM((_K, _D), jnp.float32),
            pltpu.SemaphoreType.DMA,
            pltpu.SemaphoreType.DMA,
            pltpu.SemaphoreType.DMA,
            pltpu.SemaphoreType.DMA,
            pltpu.SemaphoreType.DMA,
            pltpu.SemaphoreType.DMA,
            pltpu.SemaphoreType.DMA,
            pltpu.SemaphoreType.DMA,
        ],
    )
    def k(xh_hbm, prev_hbm, src_hbm, dst_hbm, w_hbm, out_hbm,
          acc, pidx, sidx, didx, wseg,
          rb0, rb1, rb2, rb3, g0, g1, g2, g3, s0, s1, s2, s3):
        c = lax.axis_index("c")
        s = lax.axis_index("s")
        rbufs = (rb0, rb1, rb2, rb3)
        gsems = (g0, g1, g2, g3)
        ssems = (s0, s1, s2, s3)

        pltpu.sync_copy(prev_hbm.at[s], pidx)

        def tfp(r, carry):
            for t in range(_K // 16):
                v = pidx[r, pl.ds(t * 16, 16)]
                pidx[r, pl.ds(t * 16, 16)] = v * 2 + c
            return carry

        lax.fori_loop(0, _ROW_CHUNKS, tfp, 0)
        pltpu.async_copy(xh_hbm.at[pidx.at[0]], rb0, g0)

        def p1(h, carry):
            for b in range(2):
                j = 2 * h + b
                nxt = j + 1

                @pl.when(nxt < _ROW_CHUNKS)
                def _():
                    pltpu.async_copy(xh_hbm.at[pidx.at[nxt]], rbufs[1 - b], gsems[1 - b])

                pltpu.make_async_copy(xh_hbm.at[pidx.at[j]], rbufs[b], gsems[b]).wait()
                base = s * _ROWS_PER_SUB + j * _K
                pltpu.sync_copy(rbufs[b], acc.at[pl.ds(base, _K)])
            return carry

        lax.fori_loop(0, _ROW_CHUNKS // 2, p1, 0)
        plsc.subcore_barrier()

        def mul(rows, j):
            def g_body(g, carry):
                wv = wseg[j, pl.ds(g * 16, 16)]
                for l in range(16):
                    w = wv[l]
                    e = g * 16 + l
                    for t in range(_D // 16):
                        rows[e, pl.ds(t * 16, 16)] = rows[e, pl.ds(t * 16, 16)] * w
                return carry

            lax.fori_loop(0, _K // 16, g_body, 0)

        def p2seg(gseg, carry):
            pltpu.sync_copy(src_hbm.at[s, gseg], sidx)
            pltpu.sync_copy(dst_hbm.at[s, gseg], didx)
            pltpu.sync_copy(w_hbm.at[s, gseg], wseg)

            def tfs(r, inner):
                for t in range(_K // 16):
                    v = sidx[r, pl.ds(t * 16, 16)]
                    sidx[r, pl.ds(t * 16, 16)] = v * 2 + c
                return inner

            lax.fori_loop(0, _SEG, tfs, 0)
            for b in range(_NBUF):
                pltpu.async_copy(xh_hbm.at[sidx.at[b]], rbufs[b], gsems[b])

            def ring(g, inner):
                for b in range(_NBUF):
                    k_ = g * _NBUF + b
                    pltpu.make_async_copy(xh_hbm.at[sidx.at[k_]], rbufs[b], gsems[b]).wait()
                    mul(rbufs[b], k_)
                    bp = (b - 1) % _NBUF
                    kp = k_ - 1

                    @pl.when((k_ >= 1) & (k_ + 3 < _SEG))
                    def _():
                        pltpu.make_async_copy(
                            rbufs[bp], acc.at[didx.at[kp]], ssems[bp]).wait()
                        pltpu.async_copy(
                            xh_hbm.at[sidx.at[kp + _NBUF]], rbufs[bp], gsems[bp])

                    pltpu.async_copy(rbufs[b], acc.at[didx.at[k_]], ssems[b], add=True)
                return inner

            lax.fori_loop(0, _SEG // _NBUF, ring, 0)
            for b in range(_NBUF):
                kq = _SEG - _NBUF + b
                pltpu.make_async_copy(rbufs[b], acc.at[didx.at[kq]], ssems[b]).wait()
            return carry

        lax.fori_loop(0, _NSEG, p2seg, 0)
        plsc.subcore_barrier()

        base = s * _ROWS_PER_SUB
        pltpu.sync_copy(acc.at[pl.ds(base, _ROWS_PER_SUB)],
                        out_hbm.at[pl.ds(c * _NPAD + base, _ROWS_PER_SUB)])

    return k(xh, prevp, srcp, dstp, wp)


def _tc_head(hsc, fs_w, W2, b2, W_out, b_out):
    BN = 512
    nblk = _NPAD // BN
    fsw2 = fs_w.reshape(256, 1)
    b2r = b2.reshape(1, 128)
    woutp = jnp.zeros((128, 128), jnp.float32).at[:, :40].set(W_out)
    boutp = jnp.zeros((1, 128), jnp.float32).at[0, :40].set(b_out)

    def body(h0_ref, h1_ref, g_ref, w2_ref, b2_ref, wo_ref, bo_ref, o_ref):
        g = 0.5 * jax.nn.sigmoid(g_ref[...])
        w2p = w2_ref[...] * g
        x = jnp.dot(h0_ref[...], w2p[:128, :], preferred_element_type=jnp.float32)
        x = x + jnp.dot(h1_ref[...], w2p[128:, :], preferred_element_type=jnp.float32)
        x = jnp.maximum(x + b2_ref[...], 0.0)
        o_ref[...] = jnp.dot(x, wo_ref[...], preferred_element_type=jnp.float32) + bo_ref[...]

    out = pl.pallas_call(
        body,
        grid=(nblk,),
        in_specs=[
            pl.BlockSpec((BN, _D), lambda i: (i, 0)),
            pl.BlockSpec((BN, _D), lambda i: (i + nblk, 0)),
            pl.BlockSpec((256, 1), lambda i: (0, 0)),
            pl.BlockSpec((256, 128), lambda i: (0, 0)),
            pl.BlockSpec((1, 128), lambda i: (0, 0)),
            pl.BlockSpec((128, 128), lambda i: (0, 0)),
            pl.BlockSpec((1, 128), lambda i: (0, 0)),
        ],
        out_specs=pl.BlockSpec((BN, 128), lambda i: (i, 0)),
        out_shape=jax.ShapeDtypeStruct((_NPAD, 128), jnp.float32),
    )(hsc, hsc, fsw2, W2, b2r, woutp, boutp)
    return out[:_N, :40]


def kernel(X, edge_index, edge_weight, previous_indices, sampled_nodes,
           fs_w, W1, b1, W2, b2, W_out, b_out):
    xh = X.reshape(2 * _N, _D)
    zi = jnp.zeros((_NPAD - _N,), jnp.int32)
    ze = jnp.zeros((_EPAD - _E,), jnp.int32)
    prevp = jnp.concatenate([previous_indices[2], zi]).reshape(_NS, _ROW_CHUNKS, _K)
    srcp = jnp.concatenate([edge_index[2, 0], ze]).reshape(_NS, _NSEG, _SEG, _K)
    dstp = jnp.concatenate([edge_index[2, 1], ze]).reshape(_NS, _NSEG, _SEG, _K)
    wp = jnp.concatenate([edge_weight[2], ze.astype(jnp.float32)]).reshape(
        _NS, _NSEG, _SEG, _K)
    hsc = _sc_aggregate(xh, prevp, srcp, dstp, wp)
    return _tc_head(hsc, fs_w, W2, b2, W_out, b_out)
